# Initial kernel scaffold; baseline (speedup 1.0000x reference)
import jax
import jax.numpy as jnp
from jax.experimental import pallas as pl


def kernel(node_attrs, node_feats, edge_attrs, edge_feats, edge_index, W_up, W_down, W1, W2, W3, W4, W_lin, W_skip):
    raise NotImplementedError("write your pallas kernel here")


# speedup vs baseline: 2.4970x; 2.4970x over previous
import functools
import math

import jax
import jax.numpy as jnp
from jax import lax
from jax.experimental import pallas as pl
from jax.experimental.pallas import tpu as pltpu
from jax.experimental.pallas import tpu_sc as plsc

N = 10000
E = 320000
D = 128
D_DOWN = 64
D_EDGE = 16
AVG_NEIGH = 32.0
MLP_IN = D_EDGE + 2 * D_DOWN

NC = 2
NS = 16
NW = NC * NS
CH = 128
NCHUNK = E // CH
KMAX = -(-NCHUNK // NW)

_INV_D = 1.0 / math.sqrt(D)
_INV_MLP_IN = 1.0 / math.sqrt(MLP_IN)
_INV_256 = 1.0 / 16.0


_ROWS0 = 2048


def _node_linears_body(nf_ref, wup_ref, wdown_ref, wskip_ref,
                       up_ref, down_ref, sc_ref):
    nf = nf_ref[...]
    up_ref[...] = jnp.dot(nf, wup_ref[...],
                          preferred_element_type=jnp.float32) * _INV_D
    down_ref[...] = jnp.dot(nf, wdown_ref[...],
                            preferred_element_type=jnp.float32) * _INV_D
    sc_ref[...] = jnp.dot(nf, wskip_ref[...],
                          preferred_element_type=jnp.float32) * _INV_D


def _node_linears(node_feats, W_up, W_down_pad, W_skip):
    return pl.pallas_call(
        _node_linears_body,
        grid=(pl.cdiv(N, _ROWS0),),
        in_specs=[
            pl.BlockSpec((_ROWS0, D), lambda i: (i, 0)),
            pl.BlockSpec((D, D), lambda i: (0, 0)),
            pl.BlockSpec((D, D), lambda i: (0, 0)),
            pl.BlockSpec((D, D), lambda i: (0, 0)),
        ],
        out_specs=[
            pl.BlockSpec((_ROWS0, D), lambda i: (i, 0)),
            pl.BlockSpec((_ROWS0, D), lambda i: (i, 0)),
            pl.BlockSpec((_ROWS0, D), lambda i: (i, 0)),
        ],
        out_shape=[
            jax.ShapeDtypeStruct((N, D), jnp.float32),
            jax.ShapeDtypeStruct((N, D), jnp.float32),
            jax.ShapeDtypeStruct((N, D), jnp.float32),
        ],
    )(node_feats, W_up, W_down_pad, W_skip)


def _sc_gather(sender, receiver, down, up):
    mesh = plsc.VectorSubcoreMesh(core_axis_name="c", subcore_axis_name="s",
                                  num_cores=NC, num_subcores=NS)

    @functools.partial(
        pl.kernel,
        out_type=[
            jax.ShapeDtypeStruct((E, D), jnp.float32),
            jax.ShapeDtypeStruct((E, D), jnp.float32),
            jax.ShapeDtypeStruct((E, D), jnp.float32),
        ],
        mesh=mesh,
        scratch_types=[
            pltpu.VMEM((CH,), jnp.int32),
            pltpu.VMEM((CH,), jnp.int32),
            pltpu.VMEM((CH, D), jnp.float32),
            pltpu.VMEM((CH, D), jnp.float32),
            pltpu.VMEM((CH, D), jnp.float32),
            pltpu.SemaphoreType.DMA,
        ],
    )
    def k(sender_hbm, receiver_hbm, down_hbm, up_hbm,
          ds_hbm, dr_hbm, ups_hbm,
          idx_s, idx_r, ds_v, dr_v, ups_v, sem):
        wid = lax.axis_index("s") * NC + lax.axis_index("c")

        def body(kk, carry):
            cid = wid + kk * NW

            @pl.when(cid < NCHUNK)
            def _():
                base = pl.multiple_of(cid * CH, CH)
                pltpu.sync_copy(sender_hbm.at[pl.ds(base, CH)], idx_s)
                pltpu.sync_copy(receiver_hbm.at[pl.ds(base, CH)], idx_r)
                pltpu.async_copy(down_hbm.at[idx_s], ds_v, sem).wait()
                pltpu.async_copy(down_hbm.at[idx_r], dr_v, sem).wait()
                pltpu.async_copy(up_hbm.at[idx_s], ups_v, sem).wait()
                pltpu.sync_copy(ds_v, ds_hbm.at[pl.ds(base, CH)])
                pltpu.sync_copy(dr_v, dr_hbm.at[pl.ds(base, CH)])
                pltpu.sync_copy(ups_v, ups_hbm.at[pl.ds(base, CH)])

            return carry

        lax.fori_loop(0, KMAX, body, None)

    return k(sender, receiver, down, up)


_TEDGE = 1024


def _silu(x):
    return x / (1.0 + jnp.exp(-x))


def _edge_mlp_body(ef_ref, ds_ref, dr_ref, ups_ref, ea_ref,
                   w1a_ref, w1b_ref, w2_ref, w3_ref, w4_ref, mji_ref):
    x0 = jnp.concatenate([ds_ref[...], dr_ref[...]], axis=-1)
    h = jnp.dot(ef_ref[...], w1a_ref[...], preferred_element_type=jnp.float32)
    h = h + jnp.dot(x0, w1b_ref[...], preferred_element_type=jnp.float32)
    h = _silu(h * _INV_MLP_IN)
    h = _silu(jnp.dot(h, w2_ref[...], preferred_element_type=jnp.float32) * _INV_256)
    h = _silu(jnp.dot(h, w3_ref[...], preferred_element_type=jnp.float32) * _INV_256)
    tpw = jnp.dot(h, w4_ref[...], preferred_element_type=jnp.float32) * _INV_256
    mji_ref[...] = ups_ref[...] * (ea_ref[...] * tpw)


def _edge_mlp(edge_feats, ds, dr, ups, edge_attrs, W1a, W1b, W2, W3, W4):
    return pl.pallas_call(
        _edge_mlp_body,
        grid=(pl.cdiv(E, _TEDGE),),
        in_specs=[
            pl.BlockSpec((_TEDGE, D_EDGE), lambda i: (i, 0)),
            pl.BlockSpec((_TEDGE, D), lambda i: (i, 0)),
            pl.BlockSpec((_TEDGE, D), lambda i: (i, 0)),
            pl.BlockSpec((_TEDGE, D), lambda i: (i, 0)),
            pl.BlockSpec((_TEDGE, 1), lambda i: (i, 0)),
            pl.BlockSpec((D_EDGE, 256), lambda i: (0, 0)),
            pl.BlockSpec((2 * D, 256), lambda i: (0, 0)),
            pl.BlockSpec((256, 256), lambda i: (0, 0)),
            pl.BlockSpec((256, 256), lambda i: (0, 0)),
            pl.BlockSpec((256, D), lambda i: (0, 0)),
        ],
        out_specs=pl.BlockSpec((_TEDGE, D), lambda i: (i, 0)),
        out_shape=jax.ShapeDtypeStruct((E, D), jnp.float32),
    )(edge_feats, ds, dr, ups, edge_attrs, W1a, W1b, W2, W3, W4)


def _sc_scatter(receiver, mji, zeros):
    mesh = plsc.VectorSubcoreMesh(core_axis_name="c", subcore_axis_name="s",
                                  num_cores=NC, num_subcores=NS)

    @functools.partial(
        pl.kernel,
        out_type=jax.ShapeDtypeStruct((NC, N, D), jnp.float32),
        mesh=mesh,
        scratch_types=[
            pltpu.VMEM((CH,), jnp.int32),
            pltpu.VMEM((CH, D), jnp.float32),
            pltpu.VMEM_SHARED((N, D), jnp.float32),
        ],
    )
    def k(receiver_hbm, mji_hbm, zeros_hbm, out_hbm,
          idx_r, mji_v, msg_spmem):
        c = lax.axis_index("c")
        s = lax.axis_index("s")
        wid = s * NC + c

        @pl.when(s == 0)
        def _():
            pltpu.sync_copy(zeros_hbm, msg_spmem)

        plsc.subcore_barrier()

        def body(kk, carry):
            cid = wid + kk * NW

            @pl.when(cid < NCHUNK)
            def _():
                base = pl.multiple_of(cid * CH, CH)
                pltpu.sync_copy(receiver_hbm.at[pl.ds(base, CH)], idx_r)
                pltpu.sync_copy(mji_hbm.at[pl.ds(base, CH)], mji_v)
                pltpu.sync_copy(mji_v, msg_spmem.at[idx_r], add=True)

            return carry

        lax.fori_loop(0, KMAX, body, None)

        plsc.subcore_barrier()

        @pl.when(s == 0)
        def _():
            pltpu.sync_copy(msg_spmem, out_hbm.at[c])

    return k(receiver, mji, zeros)


_ROWSD = 2048


def _finalize_body(p_ref, wlin_ref, out_ref):
    m = p_ref[0] + p_ref[1]
    out_ref[...] = jnp.dot(m, wlin_ref[...], preferred_element_type=jnp.float32) \
        * (_INV_D / AVG_NEIGH)


def _finalize(partials, W_lin):
    return pl.pallas_call(
        _finalize_body,
        grid=(pl.cdiv(N, _ROWSD),),
        in_specs=[
            pl.BlockSpec((NC, _ROWSD, D), lambda i: (0, i, 0)),
            pl.BlockSpec((D, D), lambda i: (0, 0)),
        ],
        out_specs=pl.BlockSpec((_ROWSD, D), lambda i: (i, 0)),
        out_shape=jax.ShapeDtypeStruct((N, D), jnp.float32),
    )(partials, W_lin)


def kernel(node_attrs, node_feats, edge_attrs, edge_feats, edge_index,
           W_up, W_down, W1, W2, W3, W4, W_lin, W_skip):
    sender = edge_index[0]
    receiver = edge_index[1]
    W_down_pad = jnp.pad(W_down, ((0, 0), (0, D - D_DOWN)))
    W1b_ext = jnp.concatenate([
        W1[D_EDGE:D_EDGE + D_DOWN],
        jnp.zeros((D - D_DOWN, 256), jnp.float32),
        W1[D_EDGE + D_DOWN:],
        jnp.zeros((D - D_DOWN, 256), jnp.float32),
    ], axis=0)
    up, down, sc = _node_linears(node_feats, W_up, W_down_pad, W_skip)
    ds, dr, ups = _sc_gather(sender, receiver, down, up)
    mji = _edge_mlp(edge_feats, ds, dr, ups, edge_attrs,
                    W1[:D_EDGE], W1b_ext, W2, W3, W4)
    partials = _sc_scatter(receiver, mji, jnp.zeros((N, D), jnp.float32))
    message = _finalize(partials, W_lin)
    return message.reshape(N, D, 1), sc


# speedup vs baseline: 3.8561x; 1.5443x over previous
import functools
import math

import jax
import jax.numpy as jnp
from jax import lax
from jax.experimental import pallas as pl
from jax.experimental.pallas import tpu as pltpu
from jax.experimental.pallas import tpu_sc as plsc

N = 10000
E = 320000
D = 128
D_DOWN = 64
D_EDGE = 16
AVG_NEIGH = 32.0
MLP_IN = D_EDGE + 2 * D_DOWN

NC = 2
NS = 16
NW = NC * NS
CH = 128
NCHUNK = E // CH
KW = 80
TAIL = NCHUNK - KW * (NW - 1)
TAIL_LOAD = 24
NCHUNK_PAD = KW * (NW - 1) + TAIL_LOAD

_INV_D = 1.0 / math.sqrt(D)
_INV_MLP_IN = 1.0 / math.sqrt(MLP_IN)
_INV_256 = 1.0 / 16.0


def _sc_mesh():
    return plsc.VectorSubcoreMesh(core_axis_name="c", subcore_axis_name="s",
                                  num_cores=NC, num_subcores=NS)


def _worker_span(wid):
    start = pl.multiple_of(KW * wid, KW)
    cnt = jnp.where(wid < NW - 1, KW, TAIL)
    return start, cnt


def _load_idx(src_hbm, dst_vmem, wid, start):
    @pl.when(wid < NW - 1)
    def _():
        pltpu.sync_copy(src_hbm.at[pl.ds(start, KW)], dst_vmem)

    @pl.when(wid == NW - 1)
    def _():
        pltpu.sync_copy(src_hbm.at[pl.ds(start, TAIL_LOAD)],
                        dst_vmem.at[pl.ds(0, TAIL_LOAD)])


_ROWS0 = 2048


def _rne16(x):
    bits = jax.lax.bitcast_convert_type(x, jnp.int32)
    return jax.lax.shift_right_logical(
        bits + 0x7FFF + (jax.lax.shift_right_logical(bits, 16) & 1), 16)


def _node_linears_body(nf_ref, wup_ref, wdown_ref, wskip_ref,
                       p_ref, sc_ref):
    nf = nf_ref[...]
    up = jnp.dot(nf, wup_ref[...], preferred_element_type=jnp.float32) * _INV_D
    down = jnp.dot(nf, wdown_ref[...],
                   preferred_element_type=jnp.float32) * _INV_D
    p_ref[...] = _rne16(up) | (_rne16(down) << 16)
    sc_ref[...] = jnp.dot(nf, wskip_ref[...],
                          preferred_element_type=jnp.float32) * _INV_D


def _node_linears(node_feats, W_up, W_down_pad, W_skip):
    return pl.pallas_call(
        _node_linears_body,
        grid=(pl.cdiv(N, _ROWS0),),
        in_specs=[
            pl.BlockSpec((_ROWS0, D), lambda i: (i, 0)),
            pl.BlockSpec((D, D), lambda i: (0, 0)),
            pl.BlockSpec((D, D), lambda i: (0, 0)),
            pl.BlockSpec((D, D), lambda i: (0, 0)),
        ],
        out_specs=[
            pl.BlockSpec((_ROWS0, D), lambda i: (i, 0)),
            pl.BlockSpec((_ROWS0, D), lambda i: (i, 0)),
        ],
        out_shape=[
            jax.ShapeDtypeStruct((N, D), jnp.int32),
            jax.ShapeDtypeStruct((N, D), jnp.float32),
        ],
    )(node_feats, W_up, W_down_pad, W_skip)


def _sc_gather(sender2d, recv2d, P):
    @functools.partial(
        pl.kernel,
        out_type=jax.ShapeDtypeStruct((E, D), jnp.int32),
        mesh=_sc_mesh(),
        scratch_types=[
            pltpu.VMEM((KW, CH), jnp.int32),
            pltpu.VMEM((KW, CH), jnp.int32),
            [pltpu.VMEM((CH, D), jnp.int32) for _ in range(2)],
            [pltpu.VMEM((CH, D), jnp.int32) for _ in range(2)],
            [pltpu.SemaphoreType.DMA for _ in range(2)],
            [pltpu.SemaphoreType.DMA for _ in range(2)],
        ],
    )
    def k(sender_hbm, recv_hbm, p_hbm, g_hbm,
          idx_s, idx_r, sbufs, rbufs, ssems, rsems):
        wid = lax.axis_index("s") * NC + lax.axis_index("c")
        start, cnt = _worker_span(wid)

        _load_idx(sender_hbm, idx_s, wid, start)
        _load_idx(recv_hbm, idx_r, wid, start)

        def fire(kchunk, b):
            pltpu.async_copy(p_hbm.at[idx_s.at[kchunk]], sbufs[b], ssems[b])
            pltpu.async_copy(p_hbm.at[idx_r.at[kchunk]], rbufs[b], rsems[b])

        fire(0, 0)
        fire(1, 1)

        def outer(kk, carry):
            for b in range(2):
                kchunk = kk * 2 + b

                @pl.when(kchunk < cnt)
                def _(kchunk=kchunk, b=b):
                    pltpu.make_async_copy(p_hbm.at[idx_s.at[kchunk]],
                                          sbufs[b], ssems[b]).wait()
                    pltpu.make_async_copy(p_hbm.at[idx_r.at[kchunk]],
                                          rbufs[b], rsems[b]).wait()

                    sb, rb = sbufs[b], rbufs[b]

                    def merge(e, c2):
                        for j in range(4):
                            hi = rb[e, pl.ds(j * 16, 16)] & (-65536)
                            sb[e, pl.ds(D_DOWN + j * 16, 16)] = \
                                sb[e, pl.ds(D_DOWN + j * 16, 16)] | hi
                        return c2

                    lax.fori_loop(0, CH, merge, None)
                    ebase = pl.multiple_of((start + kchunk) * CH, CH)
                    pltpu.sync_copy(sb, g_hbm.at[pl.ds(ebase, CH)])

                    @pl.when(kchunk + 2 < cnt)
                    def _():
                        fire(kchunk + 2, b)

            return carry

        lax.fori_loop(0, KW // 2, outer, None)

    return k(sender2d, recv2d, P)


_TEDGE = 1024


def _silu(x):
    return x / (1.0 + jnp.exp(-x))


def _edge_mlp_body(ef_ref, g_ref, ea_ref,
                   w1a_ref, w1b_ref, w2_ref, w3_ref, w4_ref, mji_ref):
    gi = g_ref[...]
    ups = jax.lax.bitcast_convert_type(gi << 16, jnp.float32)
    dsdr = jax.lax.bitcast_convert_type(gi & (-65536), jnp.float32)
    h = jnp.dot(ef_ref[...], w1a_ref[...], preferred_element_type=jnp.float32)
    h = h + jnp.dot(dsdr, w1b_ref[...], preferred_element_type=jnp.float32)
    h = _silu(h * _INV_MLP_IN)
    h = _silu(jnp.dot(h, w2_ref[...], preferred_element_type=jnp.float32) * _INV_256)
    h = _silu(jnp.dot(h, w3_ref[...], preferred_element_type=jnp.float32) * _INV_256)
    tpw = jnp.dot(h, w4_ref[...], preferred_element_type=jnp.float32) * _INV_256
    mji_ref[...] = ups * (ea_ref[...] * tpw)


def _edge_mlp(edge_feats, g, edge_attrs, W1a, W1b, W2, W3, W4):
    return pl.pallas_call(
        _edge_mlp_body,
        grid=(pl.cdiv(E, _TEDGE),),
        in_specs=[
            pl.BlockSpec((_TEDGE, D_EDGE), lambda i: (i, 0)),
            pl.BlockSpec((_TEDGE, D), lambda i: (i, 0)),
            pl.BlockSpec((_TEDGE, 1), lambda i: (i, 0)),
            pl.BlockSpec((D_EDGE, 256), lambda i: (0, 0)),
            pl.BlockSpec((2 * D_DOWN, 256), lambda i: (0, 0)),
            pl.BlockSpec((256, 256), lambda i: (0, 0)),
            pl.BlockSpec((256, 256), lambda i: (0, 0)),
            pl.BlockSpec((256, D), lambda i: (0, 0)),
        ],
        out_specs=pl.BlockSpec((_TEDGE, D), lambda i: (i, 0)),
        out_shape=jax.ShapeDtypeStruct((E, D), jnp.float32),
    )(edge_feats, g, edge_attrs, W1a, W1b, W2, W3, W4)


def _sc_scatter(recv2d, mji, zeros):
    @functools.partial(
        pl.kernel,
        out_type=jax.ShapeDtypeStruct((NC, N, D), jnp.float32),
        mesh=_sc_mesh(),
        scratch_types=[
            pltpu.VMEM((KW, CH), jnp.int32),
            [pltpu.VMEM((CH, D), jnp.float32) for _ in range(2)],
            pltpu.VMEM_SHARED((N, D), jnp.float32),
            [pltpu.SemaphoreType.DMA for _ in range(2)],
        ],
    )
    def k(recv_hbm, mji_hbm, zeros_hbm, out_hbm,
          idx_r, mbufs, msg_spmem, msems):
        c = lax.axis_index("c")
        s = lax.axis_index("s")
        wid = s * NC + c
        start, cnt = _worker_span(wid)

        @pl.when(s == 0)
        def _():
            pltpu.sync_copy(zeros_hbm, msg_spmem)

        _load_idx(recv_hbm, idx_r, wid, start)

        def fire(kchunk, b):
            ebase = pl.multiple_of((start + kchunk) * CH, CH)
            pltpu.async_copy(mji_hbm.at[pl.ds(ebase, CH)], mbufs[b], msems[b])

        fire(0, 0)
        fire(1, 1)

        plsc.subcore_barrier()

        def outer(kk, carry):
            for b in range(2):
                kchunk = kk * 2 + b

                @pl.when(kchunk < cnt)
                def _(kchunk=kchunk, b=b):
                    ebase = pl.multiple_of((start + kchunk) * CH, CH)
                    pltpu.make_async_copy(mji_hbm.at[pl.ds(ebase, CH)],
                                          mbufs[b], msems[b]).wait()
                    pltpu.sync_copy(mbufs[b], msg_spmem.at[idx_r.at[kchunk]],
                                    add=True)

                    @pl.when(kchunk + 2 < cnt)
                    def _():
                        fire(kchunk + 2, b)

            return carry

        lax.fori_loop(0, KW // 2, outer, None)

        plsc.subcore_barrier()

        @pl.when(s == 0)
        def _():
            pltpu.sync_copy(msg_spmem, out_hbm.at[c])

    return k(recv2d, mji, zeros)


_ROWSD = 2048


def _finalize_body(p_ref, wlin_ref, out_ref):
    m = p_ref[0] + p_ref[1]
    out_ref[...] = jnp.dot(m, wlin_ref[...], preferred_element_type=jnp.float32) \
        * (_INV_D / AVG_NEIGH)


def _finalize(partials, W_lin):
    return pl.pallas_call(
        _finalize_body,
        grid=(pl.cdiv(N, _ROWSD),),
        in_specs=[
            pl.BlockSpec((NC, _ROWSD, D), lambda i: (0, i, 0)),
            pl.BlockSpec((D, D), lambda i: (0, 0)),
        ],
        out_specs=pl.BlockSpec((_ROWSD, D), lambda i: (i, 0)),
        out_shape=jax.ShapeDtypeStruct((N, D), jnp.float32),
    )(partials, W_lin)


def kernel(node_attrs, node_feats, edge_attrs, edge_feats, edge_index,
           W_up, W_down, W1, W2, W3, W4, W_lin, W_skip):
    sender2d = jnp.pad(edge_index[0].reshape(NCHUNK, CH),
                       ((0, NCHUNK_PAD - NCHUNK), (0, 0)))
    recv2d = jnp.pad(edge_index[1].reshape(NCHUNK, CH),
                     ((0, NCHUNK_PAD - NCHUNK), (0, 0)))
    W_down_pad = jnp.pad(W_down, ((0, 0), (0, D - D_DOWN)))
    P, sc = _node_linears(node_feats, W_up, W_down_pad, W_skip)
    g = _sc_gather(sender2d, recv2d, P)
    mji = _edge_mlp(edge_feats, g, edge_attrs,
                    W1[:D_EDGE], W1[D_EDGE:], W2, W3, W4)
    partials = _sc_scatter(recv2d, mji, jnp.zeros((N, D), jnp.float32))
    message = _finalize(partials, W_lin)
    return message.reshape(N, D, 1), sc


# speedup vs baseline: 4.6131x; 1.1963x over previous
import functools
import math

import jax
import jax.numpy as jnp
from jax import lax
from jax.experimental import pallas as pl
from jax.experimental.pallas import tpu as pltpu
from jax.experimental.pallas import tpu_sc as plsc

N = 10000
E = 320000
D = 128
D_DOWN = 64
D_EDGE = 16
AVG_NEIGH = 32.0
MLP_IN = D_EDGE + 2 * D_DOWN

NC = 2
NS = 16
NW = NC * NS
CH = 128
NCHUNK = E // CH

SLICES = 4
NCHUNK_S = NCHUNK // SLICES
SLICE_E = NCHUNK_S * CH
SPAN = -(-NCHUNK_S // NW)
IDX_LOAD = 32
NCHUNK_PAD = 2528

_INV_D = 1.0 / math.sqrt(D)
_INV_MLP_IN = 1.0 / math.sqrt(MLP_IN)
_INV_256 = 1.0 / 16.0


def _sc_mesh():
    return plsc.VectorSubcoreMesh(core_axis_name="c", subcore_axis_name="s",
                                  num_cores=NC, num_subcores=NS)


def _worker_span(wid):
    start_l = SPAN * wid
    cnt = jnp.minimum(SPAN, NCHUNK_S - start_l)
    return start_l, cnt


def _load_idx(src_hbm, dst_vmem, gstart):
    off = gstart & 7
    astart = pl.multiple_of(gstart - off, 8)
    pltpu.sync_copy(src_hbm.at[pl.ds(astart, IDX_LOAD)], dst_vmem)
    return off


_ROWS0 = 2048


def _rne16(x):
    bits = jax.lax.bitcast_convert_type(x, jnp.int32)
    return jax.lax.shift_right_logical(
        bits + 0x7FFF + (jax.lax.shift_right_logical(bits, 16) & 1), 16)


def _node_linears_body(nf_ref, wup_ref, wdown_ref, wskip_ref,
                       p_ref, sc_ref):
    nf = nf_ref[...]
    up = jnp.dot(nf, wup_ref[...], preferred_element_type=jnp.float32) * _INV_D
    down = jnp.dot(nf, wdown_ref[...],
                   preferred_element_type=jnp.float32) * _INV_D
    p_ref[...] = _rne16(up) | (_rne16(down) << 16)
    sc_ref[...] = jnp.dot(nf, wskip_ref[...],
                          preferred_element_type=jnp.float32) * _INV_D


def _node_linears(node_feats, W_up, W_down_pad, W_skip):
    return pl.pallas_call(
        _node_linears_body,
        grid=(pl.cdiv(N, _ROWS0),),
        in_specs=[
            pl.BlockSpec((_ROWS0, D), lambda i: (i, 0)),
            pl.BlockSpec((D, D), lambda i: (0, 0)),
            pl.BlockSpec((D, D), lambda i: (0, 0)),
            pl.BlockSpec((D, D), lambda i: (0, 0)),
        ],
        out_specs=[
            pl.BlockSpec((_ROWS0, D), lambda i: (i, 0)),
            pl.BlockSpec((_ROWS0, D), lambda i: (i, 0)),
        ],
        out_shape=[
            jax.ShapeDtypeStruct((N, D), jnp.int32),
            jax.ShapeDtypeStruct((N, D), jnp.float32),
        ],
    )(node_feats, W_up, W_down_pad, W_skip)


def _sc_gather(sender2d, recv2d, P, sl):
    base = sl * NCHUNK_S

    @functools.partial(
        pl.kernel,
        out_type=jax.ShapeDtypeStruct((SLICE_E, D), jnp.int32),
        mesh=_sc_mesh(),
        scratch_types=[
            pltpu.VMEM((IDX_LOAD, CH), jnp.int32),
            pltpu.VMEM((IDX_LOAD, CH), jnp.int32),
            [pltpu.VMEM((CH, D), jnp.int32) for _ in range(2)],
            [pltpu.VMEM((CH, D), jnp.int32) for _ in range(2)],
            [pltpu.SemaphoreType.DMA for _ in range(2)],
            [pltpu.SemaphoreType.DMA for _ in range(2)],
        ],
        name=f"edge_gather_{sl}",
    )
    def k(sender_hbm, recv_hbm, p_hbm, g_hbm,
          idx_s, idx_r, sbufs, rbufs, ssems, rsems):
        wid = lax.axis_index("s") * NC + lax.axis_index("c")
        start_l, cnt = _worker_span(wid)

        off = _load_idx(sender_hbm, idx_s, base + start_l)
        _load_idx(recv_hbm, idx_r, base + start_l)

        def fire(kchunk, b):
            pltpu.async_copy(p_hbm.at[idx_s.at[off + kchunk]],
                             sbufs[b], ssems[b])
            pltpu.async_copy(p_hbm.at[idx_r.at[off + kchunk]],
                             rbufs[b], rsems[b])

        fire(0, 0)
        fire(1, 1)

        def outer(kk, carry):
            for b in range(2):
                kchunk = kk * 2 + b

                @pl.when(kchunk < cnt)
                def _(kchunk=kchunk, b=b):
                    pltpu.make_async_copy(p_hbm.at[idx_s.at[off + kchunk]],
                                          sbufs[b], ssems[b]).wait()
                    pltpu.make_async_copy(p_hbm.at[idx_r.at[off + kchunk]],
                                          rbufs[b], rsems[b]).wait()

                    sb, rb = sbufs[b], rbufs[b]

                    def merge(e, c2):
                        for j in range(4):
                            hi = rb[e, pl.ds(j * 16, 16)] & (-65536)
                            sb[e, pl.ds(D_DOWN + j * 16, 16)] = \
                                sb[e, pl.ds(D_DOWN + j * 16, 16)] | hi
                        return c2

                    lax.fori_loop(0, CH, merge, None)
                    ebase = pl.multiple_of((start_l + kchunk) * CH, CH)
                    pltpu.sync_copy(sb, g_hbm.at[pl.ds(ebase, CH)])

                    @pl.when(kchunk + 2 < cnt)
                    def _():
                        fire(kchunk + 2, b)

            return carry

        lax.fori_loop(0, SPAN // 2, outer, None)

    return k(sender2d, recv2d, P)


_TEDGE = 2000


def _silu(x):
    return x / (1.0 + jnp.exp(-x))


def _edge_mlp_body(ef_ref, g_ref, ea_ref,
                   w1a_ref, w1b_ref, w2_ref, w3_ref, w4_ref, mji_ref):
    gi = g_ref[...]
    ups = jax.lax.bitcast_convert_type(gi << 16, jnp.float32)
    dsdr = jax.lax.bitcast_convert_type(gi & (-65536), jnp.float32)
    h = jnp.dot(ef_ref[...], w1a_ref[...], preferred_element_type=jnp.float32)
    h = h + jnp.dot(dsdr, w1b_ref[...], preferred_element_type=jnp.float32)
    h = _silu(h * _INV_MLP_IN)
    h = _silu(jnp.dot(h, w2_ref[...], preferred_element_type=jnp.float32) * _INV_256)
    h = _silu(jnp.dot(h, w3_ref[...], preferred_element_type=jnp.float32) * _INV_256)
    tpw = jnp.dot(h, w4_ref[...], preferred_element_type=jnp.float32) * _INV_256
    mji_ref[...] = ups * (ea_ref[...] * tpw)


def _edge_mlp(edge_feats, g, edge_attrs, W1a, W1b, W2, W3, W4, sl):
    nblk = SLICE_E // _TEDGE
    off = sl * nblk
    return pl.pallas_call(
        _edge_mlp_body,
        grid=(nblk,),
        in_specs=[
            pl.BlockSpec((_TEDGE, D_EDGE), lambda i: (i + off, 0)),
            pl.BlockSpec((_TEDGE, D), lambda i: (i, 0)),
            pl.BlockSpec((_TEDGE, 1), lambda i: (i + off, 0)),
            pl.BlockSpec((D_EDGE, 256), lambda i: (0, 0)),
            pl.BlockSpec((2 * D_DOWN, 256), lambda i: (0, 0)),
            pl.BlockSpec((256, 256), lambda i: (0, 0)),
            pl.BlockSpec((256, 256), lambda i: (0, 0)),
            pl.BlockSpec((256, D), lambda i: (0, 0)),
        ],
        out_specs=pl.BlockSpec((_TEDGE, D), lambda i: (i, 0)),
        out_shape=jax.ShapeDtypeStruct((SLICE_E, D), jnp.float32),
        name=f"edge_mlp_{sl}",
    )(edge_feats, g, edge_attrs, W1a, W1b, W2, W3, W4)


def _sc_scatter(recv2d, mji, prev, sl):
    base = sl * NCHUNK_S

    @functools.partial(
        pl.kernel,
        out_type=jax.ShapeDtypeStruct((NC, N, D), jnp.float32),
        mesh=_sc_mesh(),
        scratch_types=[
            pltpu.VMEM((IDX_LOAD, CH), jnp.int32),
            [pltpu.VMEM((CH, D), jnp.float32) for _ in range(2)],
            pltpu.VMEM_SHARED((N, D), jnp.float32),
            [pltpu.SemaphoreType.DMA for _ in range(2)],
        ],
        name=f"edge_scatter_{sl}",
    )
    def k(recv_hbm, mji_hbm, prev_hbm, out_hbm,
          idx_r, mbufs, msg_spmem, msems):
        c = lax.axis_index("c")
        s = lax.axis_index("s")
        wid = s * NC + c
        start_l, cnt = _worker_span(wid)

        @pl.when(s == 0)
        def _():
            pltpu.sync_copy(prev_hbm.at[c], msg_spmem)

        off = _load_idx(recv_hbm, idx_r, base + start_l)

        def fire(kchunk, b):
            ebase = pl.multiple_of((start_l + kchunk) * CH, CH)
            pltpu.async_copy(mji_hbm.at[pl.ds(ebase, CH)], mbufs[b], msems[b])

        fire(0, 0)
        fire(1, 1)

        plsc.subcore_barrier()

        def outer(kk, carry):
            for b in range(2):
                kchunk = kk * 2 + b

                @pl.when(kchunk < cnt)
                def _(kchunk=kchunk, b=b):
                    ebase = pl.multiple_of((start_l + kchunk) * CH, CH)
                    pltpu.make_async_copy(mji_hbm.at[pl.ds(ebase, CH)],
                                          mbufs[b], msems[b]).wait()
                    pltpu.sync_copy(mbufs[b],
                                    msg_spmem.at[idx_r.at[off + kchunk]],
                                    add=True)

                    @pl.when(kchunk + 2 < cnt)
                    def _():
                        fire(kchunk + 2, b)

            return carry

        lax.fori_loop(0, SPAN // 2, outer, None)

        plsc.subcore_barrier()

        @pl.when(s == 0)
        def _():
            pltpu.sync_copy(msg_spmem, out_hbm.at[c])

    return k(recv2d, mji, prev)


_ROWSD = 2048


def _finalize_body(p_ref, wlin_ref, out_ref):
    m = p_ref[0] + p_ref[1]
    out_ref[...] = jnp.dot(m, wlin_ref[...], preferred_element_type=jnp.float32) \
        * (_INV_D / AVG_NEIGH)


def _finalize(partials, W_lin):
    return pl.pallas_call(
        _finalize_body,
        grid=(pl.cdiv(N, _ROWSD),),
        in_specs=[
            pl.BlockSpec((NC, _ROWSD, D), lambda i: (0, i, 0)),
            pl.BlockSpec((D, D), lambda i: (0, 0)),
        ],
        out_specs=pl.BlockSpec((_ROWSD, D), lambda i: (i, 0)),
        out_shape=jax.ShapeDtypeStruct((N, D), jnp.float32),
    )(partials, W_lin)


def kernel(node_attrs, node_feats, edge_attrs, edge_feats, edge_index,
           W_up, W_down, W1, W2, W3, W4, W_lin, W_skip):
    sender2d = jnp.pad(edge_index[0].reshape(NCHUNK, CH),
                       ((0, NCHUNK_PAD - NCHUNK), (0, 0)))
    recv2d = jnp.pad(edge_index[1].reshape(NCHUNK, CH),
                     ((0, NCHUNK_PAD - NCHUNK), (0, 0)))
    W_down_pad = jnp.pad(W_down, ((0, 0), (0, D - D_DOWN)))
    P, sc = _node_linears(node_feats, W_up, W_down_pad, W_skip)
    W1a, W1b = W1[:D_EDGE], W1[D_EDGE:]

    partials = jnp.zeros((NC, N, D), jnp.float32)
    for sl in range(SLICES):
        g = _sc_gather(sender2d, recv2d, P, sl)
        mji = _edge_mlp(edge_feats, g, edge_attrs, W1a, W1b, W2, W3, W4, sl)
        partials = _sc_scatter(recv2d, mji, partials, sl)

    message = _finalize(partials, W_lin)
    return message.reshape(N, D, 1), sc
